# baseline (speedup 1.0000x reference)
import jax
import jax.numpy as jnp
from jax.experimental import pallas as pl
from jax.experimental.pallas import tpu as pltpu

N = 4096
D = 256
C = 256
NC = N // C
B = 512
NB = N // B


def _body(x_ref, adj_ref, w1_ref, wn_ref, b_ref, out_ref,
          adjbf_ref, xwa_ref, xwb_ref, buf_ref, sem_ref):
    xwb_ref[...] = jnp.dot(
        x_ref[...], w1_ref[...], preferred_element_type=jnp.float32
    ).astype(jnp.bfloat16)

    def adj_dma(j):
        return pltpu.make_async_copy(
            adj_ref.at[pl.ds(j * C, C), :],
            buf_ref.at[j % 2],
            sem_ref.at[j % 2],
        )

    adj_dma(0).start()
    for j in range(NC):
        if j + 1 < NC:
            adj_dma(j + 1).start()
        adj_dma(j).wait()
        ab = buf_ref[j % 2].astype(jnp.bfloat16)
        adjbf_ref[pl.ds(j * C, C), :] = ab
        acc = jnp.dot(ab, xwb_ref[...], preferred_element_type=jnp.float32)
        h = jnp.maximum(acc + b_ref[0], 0.0).astype(jnp.bfloat16)
        xwa_ref[pl.ds(j * C, C), :] = jnp.dot(
            h, wn_ref[0], preferred_element_type=jnp.float32
        ).astype(jnp.bfloat16)

    for j in range(NB):
        r = pl.ds(j * B, B)
        acc = jnp.dot(
            adjbf_ref[r, :], xwa_ref[...], preferred_element_type=jnp.float32
        )
        h = jnp.maximum(acc + b_ref[1], 0.0).astype(jnp.bfloat16)
        xwb_ref[r, :] = jnp.dot(
            h, wn_ref[1], preferred_element_type=jnp.float32
        ).astype(jnp.bfloat16)

    for j in range(NB):
        r = pl.ds(j * B, B)
        acc = jnp.dot(
            adjbf_ref[r, :], xwb_ref[...], preferred_element_type=jnp.float32
        )
        out_ref[r, :] = jnp.maximum(acc + b_ref[2], 0.0)


@jax.jit
def kernel(x, adj, W1, b1, W2, b2, W3, b3):
    xbf = x.astype(jnp.bfloat16)
    w1 = W1.astype(jnp.bfloat16)
    wn = jnp.stack([W2, W3]).astype(jnp.bfloat16)
    b = jnp.stack([b1, b2, b3]).reshape(3, 1, D)

    return pl.pallas_call(
        _body,
        in_specs=[
            pl.BlockSpec(memory_space=pltpu.VMEM),
            pl.BlockSpec(memory_space=pl.ANY),
            pl.BlockSpec(memory_space=pltpu.VMEM),
            pl.BlockSpec(memory_space=pltpu.VMEM),
            pl.BlockSpec(memory_space=pltpu.VMEM),
        ],
        out_specs=pl.BlockSpec(memory_space=pltpu.VMEM),
        out_shape=jax.ShapeDtypeStruct((N, D), jnp.float32),
        scratch_shapes=[
            pltpu.VMEM((N, N), jnp.bfloat16),
            pltpu.VMEM((N, D), jnp.bfloat16),
            pltpu.VMEM((N, D), jnp.bfloat16),
            pltpu.VMEM((2, C, N), jnp.float32),
            pltpu.SemaphoreType.DMA((2,)),
        ],
    )(xbf, adj, w1, wn, b)


# speedup vs baseline: 1.0532x; 1.0532x over previous
import jax
import jax.numpy as jnp
from jax import lax
from jax.experimental import pallas as pl
from jax.experimental.pallas import tpu as pltpu

N = 4096
D = 256
C = 256
NPAIR = N // (2 * C)
B = 512
NB = N // B


def _body(x_ref, adj_ref, w1_ref, wn_ref, b_ref, out_ref,
          adjbf_ref, xwa_ref, xwb_ref, buf_ref, sem_ref):
    xwb_ref[...] = jnp.dot(
        x_ref[...], w1_ref[...], preferred_element_type=jnp.float32
    ).astype(jnp.bfloat16)

    for s in range(2):
        pltpu.make_async_copy(
            adj_ref.at[pl.ds(s * C, C), :], buf_ref.at[s], sem_ref.at[s]
        ).start()

    def l0_pair(j2, carry):
        for s in range(2):
            row = 2 * C * j2 + s * C
            pltpu.make_async_copy(
                adj_ref.at[pl.ds(row, C), :], buf_ref.at[s], sem_ref.at[s]
            ).wait()
            ab = buf_ref[s].astype(jnp.bfloat16)
            adjbf_ref[pl.ds(row, C), :] = ab
            acc = jnp.dot(ab, xwb_ref[...], preferred_element_type=jnp.float32)
            h = jnp.maximum(acc + b_ref[0], 0.0).astype(jnp.bfloat16)
            xwa_ref[pl.ds(row, C), :] = jnp.dot(
                h, wn_ref[0], preferred_element_type=jnp.float32
            ).astype(jnp.bfloat16)

            @pl.when(j2 < NPAIR - 1)
            def _():
                nrow = 2 * C * (j2 + 1) + s * C
                pltpu.make_async_copy(
                    adj_ref.at[pl.ds(nrow, C), :], buf_ref.at[s], sem_ref.at[s]
                ).start()
        return carry

    lax.fori_loop(0, NPAIR, l0_pair, 0)

    def l1_chunk(j, carry):
        r = pl.ds(j * B, B)
        acc = jnp.dot(
            adjbf_ref[r, :], xwa_ref[...], preferred_element_type=jnp.float32
        )
        h = jnp.maximum(acc + b_ref[1], 0.0).astype(jnp.bfloat16)
        xwb_ref[r, :] = jnp.dot(
            h, wn_ref[1], preferred_element_type=jnp.float32
        ).astype(jnp.bfloat16)
        return carry

    lax.fori_loop(0, NB, l1_chunk, 0)

    def l2_chunk(j, carry):
        r = pl.ds(j * B, B)
        acc = jnp.dot(
            adjbf_ref[r, :], xwb_ref[...], preferred_element_type=jnp.float32
        )
        out_ref[r, :] = jnp.maximum(acc + b_ref[2], 0.0)
        return carry

    lax.fori_loop(0, NB, l2_chunk, 0)


@jax.jit
def kernel(x, adj, W1, b1, W2, b2, W3, b3):
    xbf = x.astype(jnp.bfloat16)
    w1 = W1.astype(jnp.bfloat16)
    wn = jnp.stack([W2, W3]).astype(jnp.bfloat16)
    b = jnp.stack([b1, b2, b3]).reshape(3, 1, D)

    return pl.pallas_call(
        _body,
        in_specs=[
            pl.BlockSpec(memory_space=pltpu.VMEM),
            pl.BlockSpec(memory_space=pl.ANY),
            pl.BlockSpec(memory_space=pltpu.VMEM),
            pl.BlockSpec(memory_space=pltpu.VMEM),
            pl.BlockSpec(memory_space=pltpu.VMEM),
        ],
        out_specs=pl.BlockSpec(memory_space=pltpu.VMEM),
        out_shape=jax.ShapeDtypeStruct((N, D), jnp.float32),
        scratch_shapes=[
            pltpu.VMEM((N, N), jnp.bfloat16),
            pltpu.VMEM((N, D), jnp.bfloat16),
            pltpu.VMEM((N, D), jnp.bfloat16),
            pltpu.VMEM((2, C, N), jnp.float32),
            pltpu.SemaphoreType.DMA((2,)),
        ],
    )(xbf, adj, w1, wn, b)
